# Initial kernel scaffold; baseline (speedup 1.0000x reference)
import jax
import jax.numpy as jnp
from jax.experimental import pallas as pl


def kernel(aux_out, seg_out, targets, weights):
    raise NotImplementedError("write your pallas kernel here")


# speedup vs baseline: 34.8242x; 34.8242x over previous
import numpy as np
import jax
import jax.numpy as jnp
from jax.experimental import pallas as pl
from jax.experimental.pallas import tpu as pltpu

_THRESH = np.float32(0.7)
_MIN_KEPT = 100000
_AUX_N = 2 * 1 * 32 * 64 * 64
_ROWS = 2048
_THRESH_BITS = int(np.float32(0.7).view(np.int32))
_ONE_BITS = int(np.float32(1.0).view(np.int32))


def _interp_matrix(out_size, in_size):
    pos = (np.arange(out_size, dtype=np.float32) * np.float32(in_size - 1)) \
        / np.float32(out_size - 1)
    lo = np.floor(pos).astype(np.int32)
    hi = np.minimum(lo + 1, in_size - 1)
    w = (pos - lo.astype(np.float32)).astype(np.float32)
    m = np.zeros((out_size, in_size), np.float32)
    m[np.arange(out_size), lo] += np.float32(1.0) - w
    m[np.arange(out_size), hi] += w
    return m


_MD = _interp_matrix(32, 64)
_MH = _interp_matrix(64, 128)
_MW = _interp_matrix(64, 128)


def _bce(p, t, w):
    logp = jnp.maximum(jnp.log(p), -100.0)
    log1mp = jnp.maximum(jnp.log(1.0 - p), -100.0)
    return -w * (t * logp + (1.0 - t) * log1mp)


def _down(x, md, mh, mw):
    x = jax.lax.dot_general(md, x, (((1,), (0,)), ((), ())),
                            precision=jax.lax.Precision.HIGHEST)
    x = x.reshape(32, 128, 128)
    x = jax.lax.dot_general(x, mh, (((1,), (1,)), ((), ())),
                            precision=jax.lax.Precision.HIGHEST)
    x = jax.lax.dot_general(x, mw, (((1,), (1,)), ((), ())),
                            precision=jax.lax.Precision.HIGHEST)
    return x.reshape(32, 4096)


def _fused_body(p_ref, t_ref, w_ref, a_ref, md_ref, mh_ref, mw_ref, out_ref):
    i = pl.program_id(0)
    p = p_ref[0]
    t = t_ref[0]
    w = w_ref[0]
    loss = _bce(p, t, w)
    keep = p < _THRESH
    s = jnp.sum(jnp.where(keep, loss, 0.0))
    c = jnp.sum(keep.astype(jnp.float32))
    c_le = jnp.sum((p <= _THRESH).astype(jnp.float32))

    td = _down(t_ref[0], md_ref[...], mh_ref[...], mw_ref[...])
    wd = _down(w_ref[0], md_ref[...], mh_ref[...], mw_ref[...])
    aux_s = jnp.sum(_bce(a_ref[0], td, wd))

    @pl.when(i == 0)
    def _():
        out_ref[0, 0] = 0.0
        out_ref[0, 1] = 0.0
        out_ref[0, 2] = 0.0
        out_ref[1, 0] = 0.0

    out_ref[0, 0] += s
    out_ref[0, 1] += c
    out_ref[0, 2] += aux_s
    out_ref[1, 0] += c_le


def _sel_body(p_ref, out_ref):
    k1 = jnp.int32(_MIN_KEPT + 1)

    def cond(st):
        lo, hi = st
        return lo < hi

    def body(st):
        lo, hi = st
        mid = (lo + hi) // 2
        pb = jax.lax.bitcast_convert_type(p_ref[...], jnp.int32)
        c = jnp.sum((pb <= mid).astype(jnp.int32))
        pred = c >= k1
        return (jnp.where(pred, lo, mid + 1), jnp.where(pred, hi, mid))

    lo, _ = jax.lax.while_loop(
        cond, body, (jnp.int32(_THRESH_BITS + 1), jnp.int32(_ONE_BITS)))
    out_ref[0, 0] = jax.lax.bitcast_convert_type(lo, jnp.float32)


def _resum_body(th_ref, p_ref, t_ref, w_ref, out_ref):
    i = pl.program_id(0)
    th = th_ref[0, 0]
    p = p_ref[...]
    loss = _bce(p, t_ref[...], w_ref[...])
    keep = p < th
    s = jnp.sum(jnp.where(keep, loss, 0.0))
    c = jnp.sum(keep.astype(jnp.float32))

    @pl.when(i == 0)
    def _():
        out_ref[0, 0] = 0.0
        out_ref[0, 1] = 0.0

    out_ref[0, 0] += s
    out_ref[0, 1] += c


def kernel(aux_out, seg_out, targets, weights):
    p3 = seg_out.reshape(2, 64, 16384)
    t3 = targets.reshape(2, 64, 16384)
    w3 = weights.reshape(2, 64, 16384)
    a3 = aux_out.reshape(2, 32, 4096)

    sums = pl.pallas_call(
        _fused_body,
        grid=(2,),
        out_shape=jax.ShapeDtypeStruct((2, 3), jnp.float32),
        in_specs=[
            pl.BlockSpec((1, 64, 16384), lambda i: (i, 0, 0)),
            pl.BlockSpec((1, 64, 16384), lambda i: (i, 0, 0)),
            pl.BlockSpec((1, 64, 16384), lambda i: (i, 0, 0)),
            pl.BlockSpec((1, 32, 4096), lambda i: (i, 0, 0)),
            pl.BlockSpec((32, 64), lambda i: (0, 0)),
            pl.BlockSpec((64, 128), lambda i: (0, 0)),
            pl.BlockSpec((64, 128), lambda i: (0, 0)),
        ],
        out_specs=pl.BlockSpec((2, 3), lambda i: (0, 0),
                               memory_space=pltpu.SMEM),
        compiler_params=pltpu.CompilerParams(
            dimension_semantics=("arbitrary",)),
    )(p3, t3, w3, a3, jnp.asarray(_MD), jnp.asarray(_MH), jnp.asarray(_MW))

    s_fast = sums[0, 0]
    c_fast = sums[0, 1]
    aux_sum = sums[0, 2]
    c_le = sums[1, 0]

    p2 = seg_out.reshape(_ROWS, 1024)
    t2 = targets.reshape(_ROWS, 1024)
    w2 = weights.reshape(_ROWS, 1024)

    def rare_path(_):
        thresh = pl.pallas_call(
            _sel_body,
            out_shape=jax.ShapeDtypeStruct((1, 1), jnp.float32),
            in_specs=[pl.BlockSpec((_ROWS, 1024), lambda: (0, 0))],
            out_specs=pl.BlockSpec(memory_space=pltpu.SMEM),
        )(p2)
        rows_blk = 256
        seg_sums = pl.pallas_call(
            _resum_body,
            grid=(_ROWS // rows_blk,),
            out_shape=jax.ShapeDtypeStruct((1, 2), jnp.float32),
            in_specs=[
                pl.BlockSpec(memory_space=pltpu.SMEM),
                pl.BlockSpec((rows_blk, 1024), lambda i: (i, 0)),
                pl.BlockSpec((rows_blk, 1024), lambda i: (i, 0)),
                pl.BlockSpec((rows_blk, 1024), lambda i: (i, 0)),
            ],
            out_specs=pl.BlockSpec((1, 2), lambda i: (0, 0),
                                   memory_space=pltpu.SMEM),
            compiler_params=pltpu.CompilerParams(
                dimension_semantics=("arbitrary",)),
        )(thresh, p2, t2, w2)
        return seg_sums[0, 0], seg_sums[0, 1]

    def fast_path(_):
        return s_fast, c_fast

    s, c = jax.lax.cond(c_le < np.float32(_MIN_KEPT + 1),
                        rare_path, fast_path, None)
    seg_loss = s / jnp.maximum(c, 1.0)
    aux_loss = (aux_sum / np.float32(_AUX_N))
    return seg_loss + 0.5 * aux_loss


# speedup vs baseline: 42.9366x; 1.2330x over previous
import numpy as np
import jax
import jax.numpy as jnp
from jax.experimental import pallas as pl
from jax.experimental.pallas import tpu as pltpu

_THRESH = np.float32(0.7)
_MIN_KEPT = 100000
_AUX_N = 2 * 1 * 32 * 64 * 64
_ROWS = 2048
_THRESH_BITS = int(np.float32(0.7).view(np.int32))
_ONE_BITS = int(np.float32(1.0).view(np.int32))


def _interp_matrix(out_size, in_size):
    pos = (np.arange(out_size, dtype=np.float32) * np.float32(in_size - 1)) \
        / np.float32(out_size - 1)
    lo = np.floor(pos).astype(np.int32)
    hi = np.minimum(lo + 1, in_size - 1)
    w = (pos - lo.astype(np.float32)).astype(np.float32)
    m = np.zeros((out_size, in_size), np.float32)
    m[np.arange(out_size), lo] += np.float32(1.0) - w
    m[np.arange(out_size), hi] += w
    return m


_MD = _interp_matrix(32, 64)
_MH = _interp_matrix(64, 128)
_MW = _interp_matrix(64, 128)


def _bce(p, t, w):
    logp = jnp.maximum(jnp.log(p), -100.0)
    log1mp = jnp.maximum(jnp.log(1.0 - p), -100.0)
    return -w * (t * logp + (1.0 - t) * log1mp)


def _down(x, md, mh, mwt):
    x = jax.lax.dot_general(md, x, (((1,), (0,)), ((), ())),
                            precision=jax.lax.Precision.HIGHEST)
    x = x.reshape(32, 128, 128)
    x = jax.lax.dot_general(mh, x, (((1,), (1,)), ((), ())),
                            precision=jax.lax.Precision.HIGHEST)
    x = x.reshape(2048, 128)
    x = jax.lax.dot_general(x, mwt, (((1,), (0,)), ((), ())),
                            precision=jax.lax.Precision.HIGHEST)
    return x


def _fused_body(p_ref, t_ref, w_ref, a_ref, md_ref, mh_ref, mwt_ref, out_ref):
    i = pl.program_id(0)
    p = p_ref[0]
    t = t_ref[0]
    w = w_ref[0]
    loss = _bce(p, t, w)
    keep = p < _THRESH
    s = jnp.sum(jnp.where(keep, loss, 0.0))
    c = jnp.sum(keep.astype(jnp.float32))
    c_le = jnp.sum((p <= _THRESH).astype(jnp.float32))

    td = _down(t_ref[0], md_ref[...], mh_ref[...], mwt_ref[...])
    wd = _down(w_ref[0], md_ref[...], mh_ref[...], mwt_ref[...])
    aux_s = jnp.sum(_bce(a_ref[0], td, wd))

    @pl.when(i == 0)
    def _():
        out_ref[0, 0] = 0.0
        out_ref[0, 1] = 0.0
        out_ref[0, 2] = 0.0
        out_ref[1, 0] = 0.0

    out_ref[0, 0] += s
    out_ref[0, 1] += c
    out_ref[0, 2] += aux_s
    out_ref[1, 0] += c_le


def _sel_body(p_ref, out_ref):
    k1 = jnp.int32(_MIN_KEPT + 1)

    def cond(st):
        lo, hi = st
        return lo < hi

    def body(st):
        lo, hi = st
        mid = (lo + hi) // 2
        pb = jax.lax.bitcast_convert_type(p_ref[...], jnp.int32)
        c = jnp.sum((pb <= mid).astype(jnp.int32))
        pred = c >= k1
        return (jnp.where(pred, lo, mid + 1), jnp.where(pred, hi, mid))

    lo, _ = jax.lax.while_loop(
        cond, body, (jnp.int32(_THRESH_BITS + 1), jnp.int32(_ONE_BITS)))
    out_ref[0, 0] = jax.lax.bitcast_convert_type(lo, jnp.float32)


def _resum_body(th_ref, p_ref, t_ref, w_ref, out_ref):
    i = pl.program_id(0)
    th = th_ref[0, 0]
    p = p_ref[...]
    loss = _bce(p, t_ref[...], w_ref[...])
    keep = p < th
    s = jnp.sum(jnp.where(keep, loss, 0.0))
    c = jnp.sum(keep.astype(jnp.float32))

    @pl.when(i == 0)
    def _():
        out_ref[0, 0] = 0.0
        out_ref[0, 1] = 0.0

    out_ref[0, 0] += s
    out_ref[0, 1] += c


def kernel(aux_out, seg_out, targets, weights):
    p3 = seg_out.reshape(2, 64, 16384)
    t3 = targets.reshape(2, 64, 16384)
    w3 = weights.reshape(2, 64, 16384)
    a3 = aux_out.reshape(2, 32, 64, 64).transpose(0, 2, 1, 3).reshape(2, 2048, 64)

    sums = pl.pallas_call(
        _fused_body,
        grid=(2,),
        out_shape=jax.ShapeDtypeStruct((2, 3), jnp.float32),
        in_specs=[
            pl.BlockSpec((1, 64, 16384), lambda i: (i, 0, 0)),
            pl.BlockSpec((1, 64, 16384), lambda i: (i, 0, 0)),
            pl.BlockSpec((1, 64, 16384), lambda i: (i, 0, 0)),
            pl.BlockSpec((1, 2048, 64), lambda i: (i, 0, 0)),
            pl.BlockSpec((32, 64), lambda i: (0, 0)),
            pl.BlockSpec((64, 128), lambda i: (0, 0)),
            pl.BlockSpec((128, 64), lambda i: (0, 0)),
        ],
        out_specs=pl.BlockSpec((2, 3), lambda i: (0, 0),
                               memory_space=pltpu.SMEM),
        compiler_params=pltpu.CompilerParams(
            dimension_semantics=("arbitrary",)),
    )(p3, t3, w3, a3, jnp.asarray(_MD), jnp.asarray(_MH),
      jnp.asarray(_MW.T.copy()))

    s_fast = sums[0, 0]
    c_fast = sums[0, 1]
    aux_sum = sums[0, 2]
    c_le = sums[1, 0]

    p2 = seg_out.reshape(_ROWS, 1024)
    t2 = targets.reshape(_ROWS, 1024)
    w2 = weights.reshape(_ROWS, 1024)

    def rare_path(_):
        thresh = pl.pallas_call(
            _sel_body,
            out_shape=jax.ShapeDtypeStruct((1, 1), jnp.float32),
            in_specs=[pl.BlockSpec((_ROWS, 1024), lambda: (0, 0))],
            out_specs=pl.BlockSpec(memory_space=pltpu.SMEM),
        )(p2)
        rows_blk = 256
        seg_sums = pl.pallas_call(
            _resum_body,
            grid=(_ROWS // rows_blk,),
            out_shape=jax.ShapeDtypeStruct((1, 2), jnp.float32),
            in_specs=[
                pl.BlockSpec(memory_space=pltpu.SMEM),
                pl.BlockSpec((rows_blk, 1024), lambda i: (i, 0)),
                pl.BlockSpec((rows_blk, 1024), lambda i: (i, 0)),
                pl.BlockSpec((rows_blk, 1024), lambda i: (i, 0)),
            ],
            out_specs=pl.BlockSpec((1, 2), lambda i: (0, 0),
                                   memory_space=pltpu.SMEM),
            compiler_params=pltpu.CompilerParams(
                dimension_semantics=("arbitrary",)),
        )(thresh, p2, t2, w2)
        return seg_sums[0, 0], seg_sums[0, 1]

    def fast_path(_):
        return s_fast, c_fast

    s, c = jax.lax.cond(c_le < np.float32(_MIN_KEPT + 1),
                        rare_path, fast_path, None)
    seg_loss = s / jnp.maximum(c, 1.0)
    aux_loss = (aux_sum / np.float32(_AUX_N))
    return seg_loss + 0.5 * aux_loss


# speedup vs baseline: 52.4463x; 1.2215x over previous
import numpy as np
import jax
import jax.numpy as jnp
from jax.experimental import pallas as pl
from jax.experimental.pallas import tpu as pltpu

_THRESH = np.float32(0.7)
_MIN_KEPT = 100000
_AUX_N = 2 * 1 * 32 * 64 * 64
_ROWS = 2048
_THRESH_BITS = int(np.float32(0.7).view(np.int32))
_ONE_BITS = int(np.float32(1.0).view(np.int32))


def _interp_matrix(out_size, in_size):
    pos = (np.arange(out_size, dtype=np.float32) * np.float32(in_size - 1)) \
        / np.float32(out_size - 1)
    lo = np.floor(pos).astype(np.int32)
    hi = np.minimum(lo + 1, in_size - 1)
    w = (pos - lo.astype(np.float32)).astype(np.float32)
    m = np.zeros((out_size, in_size), np.float32)
    m[np.arange(out_size), lo] += np.float32(1.0) - w
    m[np.arange(out_size), hi] += w
    return m


_MD = _interp_matrix(32, 64)
_MH = _interp_matrix(64, 128)
_MW = _interp_matrix(64, 128)


def _bce(p, t, w):
    logp = jnp.maximum(jnp.log(p), -100.0)
    log1mp = jnp.maximum(jnp.log(1.0 - p), -100.0)
    return -w * (t * logp + (1.0 - t) * log1mp)


def _down(x, md, mh, mwt):
    x = jax.lax.dot_general(md, x, (((1,), (0,)), ((), ())),
                            precision=jax.lax.Precision.DEFAULT)
    x = x.reshape(32, 128, 128)
    x = jax.lax.dot_general(mh, x, (((1,), (1,)), ((), ())),
                            precision=jax.lax.Precision.DEFAULT)
    x = x.reshape(2048, 128)
    x = jax.lax.dot_general(x, mwt, (((1,), (0,)), ((), ())),
                            precision=jax.lax.Precision.DEFAULT)
    return x


def _fused_body(p_ref, t_ref, w_ref, a_ref, md_ref, mh_ref, mwt_ref, out_ref):
    i = pl.program_id(0)
    p = p_ref[0]
    t = t_ref[0]
    w = w_ref[0]
    loss = _bce(p, t, w)
    keep = p < _THRESH
    s = jnp.sum(jnp.where(keep, loss, 0.0))
    c = jnp.sum(keep.astype(jnp.float32))
    c_le = jnp.sum((p <= _THRESH).astype(jnp.float32))

    td = _down(t_ref[0], md_ref[...], mh_ref[...], mwt_ref[...])
    wd = _down(w_ref[0], md_ref[...], mh_ref[...], mwt_ref[...])
    aux_s = jnp.sum(_bce(a_ref[0], td, wd))

    @pl.when(i == 0)
    def _():
        out_ref[0, 0] = 0.0
        out_ref[0, 1] = 0.0
        out_ref[0, 2] = 0.0
        out_ref[1, 0] = 0.0

    out_ref[0, 0] += s
    out_ref[0, 1] += c
    out_ref[0, 2] += aux_s
    out_ref[1, 0] += c_le


def _sel_body(p_ref, out_ref):
    k1 = jnp.int32(_MIN_KEPT + 1)

    def cond(st):
        lo, hi = st
        return lo < hi

    def body(st):
        lo, hi = st
        mid = (lo + hi) // 2
        pb = jax.lax.bitcast_convert_type(p_ref[...], jnp.int32)
        c = jnp.sum((pb <= mid).astype(jnp.int32))
        pred = c >= k1
        return (jnp.where(pred, lo, mid + 1), jnp.where(pred, hi, mid))

    lo, _ = jax.lax.while_loop(
        cond, body, (jnp.int32(_THRESH_BITS + 1), jnp.int32(_ONE_BITS)))
    out_ref[0, 0] = jax.lax.bitcast_convert_type(lo, jnp.float32)


def _resum_body(th_ref, p_ref, t_ref, w_ref, out_ref):
    i = pl.program_id(0)
    th = th_ref[0, 0]
    p = p_ref[...]
    loss = _bce(p, t_ref[...], w_ref[...])
    keep = p < th
    s = jnp.sum(jnp.where(keep, loss, 0.0))
    c = jnp.sum(keep.astype(jnp.float32))

    @pl.when(i == 0)
    def _():
        out_ref[0, 0] = 0.0
        out_ref[0, 1] = 0.0

    out_ref[0, 0] += s
    out_ref[0, 1] += c


def kernel(aux_out, seg_out, targets, weights):
    p3 = seg_out.reshape(2, 64, 16384)
    t3 = targets.reshape(2, 64, 16384)
    w3 = weights.reshape(2, 64, 16384)
    a3 = aux_out.reshape(2, 32, 64, 64).transpose(0, 2, 1, 3).reshape(2, 2048, 64)

    sums = pl.pallas_call(
        _fused_body,
        grid=(2,),
        out_shape=jax.ShapeDtypeStruct((2, 3), jnp.float32),
        in_specs=[
            pl.BlockSpec((1, 64, 16384), lambda i: (i, 0, 0)),
            pl.BlockSpec((1, 64, 16384), lambda i: (i, 0, 0)),
            pl.BlockSpec((1, 64, 16384), lambda i: (i, 0, 0)),
            pl.BlockSpec((1, 2048, 64), lambda i: (i, 0, 0)),
            pl.BlockSpec((32, 64), lambda i: (0, 0)),
            pl.BlockSpec((64, 128), lambda i: (0, 0)),
            pl.BlockSpec((128, 64), lambda i: (0, 0)),
        ],
        out_specs=pl.BlockSpec((2, 3), lambda i: (0, 0),
                               memory_space=pltpu.SMEM),
        compiler_params=pltpu.CompilerParams(
            dimension_semantics=("arbitrary",)),
    )(p3, t3, w3, a3, jnp.asarray(_MD), jnp.asarray(_MH),
      jnp.asarray(_MW.T.copy()))

    s_fast = sums[0, 0]
    c_fast = sums[0, 1]
    aux_sum = sums[0, 2]
    c_le = sums[1, 0]

    p2 = seg_out.reshape(_ROWS, 1024)
    t2 = targets.reshape(_ROWS, 1024)
    w2 = weights.reshape(_ROWS, 1024)

    def rare_path(_):
        thresh = pl.pallas_call(
            _sel_body,
            out_shape=jax.ShapeDtypeStruct((1, 1), jnp.float32),
            in_specs=[pl.BlockSpec((_ROWS, 1024), lambda: (0, 0))],
            out_specs=pl.BlockSpec(memory_space=pltpu.SMEM),
        )(p2)
        rows_blk = 256
        seg_sums = pl.pallas_call(
            _resum_body,
            grid=(_ROWS // rows_blk,),
            out_shape=jax.ShapeDtypeStruct((1, 2), jnp.float32),
            in_specs=[
                pl.BlockSpec(memory_space=pltpu.SMEM),
                pl.BlockSpec((rows_blk, 1024), lambda i: (i, 0)),
                pl.BlockSpec((rows_blk, 1024), lambda i: (i, 0)),
                pl.BlockSpec((rows_blk, 1024), lambda i: (i, 0)),
            ],
            out_specs=pl.BlockSpec((1, 2), lambda i: (0, 0),
                                   memory_space=pltpu.SMEM),
            compiler_params=pltpu.CompilerParams(
                dimension_semantics=("arbitrary",)),
        )(thresh, p2, t2, w2)
        return seg_sums[0, 0], seg_sums[0, 1]

    def fast_path(_):
        return s_fast, c_fast

    s, c = jax.lax.cond(c_le < np.float32(_MIN_KEPT + 1),
                        rare_path, fast_path, None)
    seg_loss = s / jnp.maximum(c, 1.0)
    aux_loss = (aux_sum / np.float32(_AUX_N))
    return seg_loss + 0.5 * aux_loss


# speedup vs baseline: 73.1079x; 1.3940x over previous
import numpy as np
import jax
import jax.numpy as jnp
from jax.experimental import pallas as pl
from jax.experimental.pallas import tpu as pltpu

_THRESH = np.float32(0.7)
_MIN_KEPT = 100000
_AUX_N = 2 * 1 * 32 * 64 * 64
_ROWS = 2048
_THRESH_BITS = int(np.float32(0.7).view(np.int32))
_ONE_BITS = int(np.float32(1.0).view(np.int32))
_DCH = 4


def _interp_matrix(out_size, in_size):
    pos = (np.arange(out_size, dtype=np.float32) * np.float32(in_size - 1)) \
        / np.float32(out_size - 1)
    lo = np.floor(pos).astype(np.int32)
    hi = np.minimum(lo + 1, in_size - 1)
    w = (pos - lo.astype(np.float32)).astype(np.float32)
    m = np.zeros((out_size, in_size), np.float32)
    m[np.arange(out_size), lo] += np.float32(1.0) - w
    m[np.arange(out_size), hi] += w
    return m


_MD = _interp_matrix(32, 64)
_MH = _interp_matrix(64, 128)
_MW = _interp_matrix(64, 128)
_MDC = _MD.reshape(32, _DCH, 16).transpose(1, 0, 2).copy()


def _bce(p, t, w):
    logp = jnp.maximum(jnp.log(p), -100.0)
    log1mp = jnp.maximum(jnp.log(1.0 - p), -100.0)
    return -w * (t * logp + (1.0 - t) * log1mp)


def _fused_body(p_ref, t_ref, w_ref, a_ref, mdc_ref, mh_ref, mwt_ref,
                out_ref, acct_ref, accw_ref):
    b = pl.program_id(0)
    c = pl.program_id(1)
    p = p_ref[0, 0]
    t = t_ref[0, 0]
    w = w_ref[0, 0]
    loss = _bce(p, t, w)
    keep = p < _THRESH
    s = jnp.sum(jnp.where(keep, loss, 0.0))
    cnt = jnp.sum(keep.astype(jnp.float32))
    c_le = jnp.sum((p <= _THRESH).astype(jnp.float32))

    mdc = mdc_ref[0]
    pd_t = jax.lax.dot_general(mdc, t, (((1,), (0,)), ((), ())),
                               precision=jax.lax.Precision.DEFAULT)
    pd_w = jax.lax.dot_general(mdc, w, (((1,), (0,)), ((), ())),
                               precision=jax.lax.Precision.DEFAULT)

    @pl.when(c == 0)
    def _():
        acct_ref[...] = pd_t
        accw_ref[...] = pd_w

    @pl.when(c != 0)
    def _():
        acct_ref[...] += pd_t
        accw_ref[...] += pd_w

    @pl.when(jnp.logical_and(b == 0, c == 0))
    def _():
        out_ref[0, 0] = 0.0
        out_ref[0, 1] = 0.0
        out_ref[0, 2] = 0.0
        out_ref[1, 0] = 0.0

    out_ref[0, 0] += s
    out_ref[0, 1] += cnt
    out_ref[1, 0] += c_le

    @pl.when(c == _DCH - 1)
    def _():
        mh = mh_ref[...]
        mwt = mwt_ref[...]

        def rest(x):
            x = jax.lax.dot_general(mh, x, (((1,), (1,)), ((), ())),
                                    precision=jax.lax.Precision.DEFAULT)
            x = jax.lax.dot_general(x, mwt, (((2,), (0,)), ((), ())),
                                    precision=jax.lax.Precision.DEFAULT)
            return x

        td = rest(acct_ref[...])
        wd = rest(accw_ref[...])
        out_ref[0, 2] += jnp.sum(_bce(a_ref[0], td, wd))


def _sel_body(p_ref, out_ref):
    k1 = jnp.int32(_MIN_KEPT + 1)

    def cond(st):
        lo, hi = st
        return lo < hi

    def body(st):
        lo, hi = st
        mid = (lo + hi) // 2
        pb = jax.lax.bitcast_convert_type(p_ref[...], jnp.int32)
        cq = jnp.sum((pb <= mid).astype(jnp.int32))
        pred = cq >= k1
        return (jnp.where(pred, lo, mid + 1), jnp.where(pred, hi, mid))

    lo, _ = jax.lax.while_loop(
        cond, body, (jnp.int32(_THRESH_BITS + 1), jnp.int32(_ONE_BITS)))
    out_ref[0, 0] = jax.lax.bitcast_convert_type(lo, jnp.float32)


def _resum_body(th_ref, p_ref, t_ref, w_ref, out_ref):
    i = pl.program_id(0)
    th = th_ref[0, 0]
    p = p_ref[...]
    loss = _bce(p, t_ref[...], w_ref[...])
    keep = p < th
    s = jnp.sum(jnp.where(keep, loss, 0.0))
    cnt = jnp.sum(keep.astype(jnp.float32))

    @pl.when(i == 0)
    def _():
        out_ref[0, 0] = 0.0
        out_ref[0, 1] = 0.0

    out_ref[0, 0] += s
    out_ref[0, 1] += cnt


def kernel(aux_out, seg_out, targets, weights):
    a4 = aux_out.reshape(2, 32, 64, 64).transpose(0, 2, 1, 3)

    sums = pl.pallas_call(
        _fused_body,
        grid=(2, _DCH),
        out_shape=jax.ShapeDtypeStruct((2, 3), jnp.float32),
        in_specs=[
            pl.BlockSpec((1, 1, 16, 128, 128), lambda b, c: (b, 0, c, 0, 0)),
            pl.BlockSpec((1, 1, 16, 128, 128), lambda b, c: (b, 0, c, 0, 0)),
            pl.BlockSpec((1, 1, 16, 128, 128), lambda b, c: (b, 0, c, 0, 0)),
            pl.BlockSpec((1, 64, 32, 64), lambda b, c: (b, 0, 0, 0)),
            pl.BlockSpec((1, 32, 16), lambda b, c: (c, 0, 0)),
            pl.BlockSpec((64, 128), lambda b, c: (0, 0)),
            pl.BlockSpec((128, 64), lambda b, c: (0, 0)),
        ],
        out_specs=pl.BlockSpec((2, 3), lambda b, c: (0, 0),
                               memory_space=pltpu.SMEM),
        scratch_shapes=[
            pltpu.VMEM((32, 128, 128), jnp.float32),
            pltpu.VMEM((32, 128, 128), jnp.float32),
        ],
        compiler_params=pltpu.CompilerParams(
            dimension_semantics=("arbitrary", "arbitrary")),
    )(seg_out, targets, weights, a4, jnp.asarray(_MDC), jnp.asarray(_MH),
      jnp.asarray(_MW.T.copy()))

    s_fast = sums[0, 0]
    c_fast = sums[0, 1]
    aux_sum = sums[0, 2]
    c_le = sums[1, 0]

    def rare_path(_):
        p2 = seg_out.reshape(_ROWS, 1024)
        t2 = targets.reshape(_ROWS, 1024)
        w2 = weights.reshape(_ROWS, 1024)
        thresh = pl.pallas_call(
            _sel_body,
            out_shape=jax.ShapeDtypeStruct((1, 1), jnp.float32),
            in_specs=[pl.BlockSpec((_ROWS, 1024), lambda: (0, 0))],
            out_specs=pl.BlockSpec(memory_space=pltpu.SMEM),
        )(p2)
        rows_blk = 256
        seg_sums = pl.pallas_call(
            _resum_body,
            grid=(_ROWS // rows_blk,),
            out_shape=jax.ShapeDtypeStruct((1, 2), jnp.float32),
            in_specs=[
                pl.BlockSpec(memory_space=pltpu.SMEM),
                pl.BlockSpec((rows_blk, 1024), lambda i: (i, 0)),
                pl.BlockSpec((rows_blk, 1024), lambda i: (i, 0)),
                pl.BlockSpec((rows_blk, 1024), lambda i: (i, 0)),
            ],
            out_specs=pl.BlockSpec((1, 2), lambda i: (0, 0),
                                   memory_space=pltpu.SMEM),
            compiler_params=pltpu.CompilerParams(
                dimension_semantics=("arbitrary",)),
        )(thresh, p2, t2, w2)
        return seg_sums[0, 0], seg_sums[0, 1]

    def fast_path(_):
        return s_fast, c_fast

    s, c = jax.lax.cond(c_le < np.float32(_MIN_KEPT + 1),
                        rare_path, fast_path, None)
    seg_loss = s / jnp.maximum(c, 1.0)
    aux_loss = aux_sum / np.float32(_AUX_N)
    return seg_loss + 0.5 * aux_loss


# speedup vs baseline: 94.9464x; 1.2987x over previous
import numpy as np
import jax
import jax.numpy as jnp
from jax.experimental import pallas as pl
from jax.experimental.pallas import tpu as pltpu

_THRESH = np.float32(0.7)
_MIN_KEPT = 100000
_AUX_N = 2 * 1 * 32 * 64 * 64
_ROWS = 2048
_THRESH_BITS = int(np.float32(0.7).view(np.int32))
_ONE_BITS = int(np.float32(1.0).view(np.int32))
_HCH = 2


def _interp_matrix(out_size, in_size):
    pos = (np.arange(out_size, dtype=np.float32) * np.float32(in_size - 1)) \
        / np.float32(out_size - 1)
    lo = np.floor(pos).astype(np.int32)
    hi = np.minimum(lo + 1, in_size - 1)
    w = (pos - lo.astype(np.float32)).astype(np.float32)
    m = np.zeros((out_size, in_size), np.float32)
    m[np.arange(out_size), lo] += np.float32(1.0) - w
    m[np.arange(out_size), hi] += w
    return m


_MD = _interp_matrix(32, 64)
_MH = _interp_matrix(64, 128)
_MW = _interp_matrix(64, 128)


def _bce(p, t, w):
    logp = jnp.maximum(jnp.log(p), -100.0)
    log1mp = jnp.maximum(jnp.log(1.0 - p), -100.0)
    return -w * (t * logp + (1.0 - t) * log1mp)


def _fused_body(p_ref, t_ref, w_ref, a_ref, md_ref, mh_ref, mwt_ref,
                out_ref, acct_ref, accw_ref):
    b = pl.program_id(0)
    c = pl.program_id(1)
    p = p_ref[0, 0]
    t = t_ref[0, 0]
    w = w_ref[0, 0]
    loss = _bce(p, t, w)
    keep = p < _THRESH
    s = jnp.sum(jnp.where(keep, loss, 0.0))
    cnt = jnp.sum(keep.astype(jnp.float32))
    c_le = jnp.sum((p <= _THRESH).astype(jnp.float32))

    md = md_ref[...]
    pd_t = jax.lax.dot_general(md, t, (((1,), (0,)), ((), ())),
                               precision=jax.lax.Precision.DEFAULT)
    pd_w = jax.lax.dot_general(md, w, (((1,), (0,)), ((), ())),
                               precision=jax.lax.Precision.DEFAULT)
    acct_ref[:, pl.ds(c * 64, 64), :] = pd_t
    accw_ref[:, pl.ds(c * 64, 64), :] = pd_w

    @pl.when(jnp.logical_and(b == 0, c == 0))
    def _():
        out_ref[0, 0] = 0.0
        out_ref[0, 1] = 0.0
        out_ref[0, 2] = 0.0
        out_ref[1, 0] = 0.0

    out_ref[0, 0] += s
    out_ref[0, 1] += cnt
    out_ref[1, 0] += c_le

    @pl.when(c == _HCH - 1)
    def _():
        mh = mh_ref[...]
        mwt = mwt_ref[...]

        def rest(x):
            x = jax.lax.dot_general(mh, x, (((1,), (1,)), ((), ())),
                                    precision=jax.lax.Precision.DEFAULT)
            x = jax.lax.dot_general(x, mwt, (((2,), (0,)), ((), ())),
                                    precision=jax.lax.Precision.DEFAULT)
            return x

        td = rest(acct_ref[...])
        wd = rest(accw_ref[...])
        out_ref[0, 2] += jnp.sum(_bce(a_ref[0], td, wd))


def _sel_body(p_ref, out_ref):
    k1 = jnp.int32(_MIN_KEPT + 1)

    def cond(st):
        lo, hi = st
        return lo < hi

    def body(st):
        lo, hi = st
        mid = (lo + hi) // 2
        pb = jax.lax.bitcast_convert_type(p_ref[...], jnp.int32)
        cq = jnp.sum((pb <= mid).astype(jnp.int32))
        pred = cq >= k1
        return (jnp.where(pred, lo, mid + 1), jnp.where(pred, hi, mid))

    lo, _ = jax.lax.while_loop(
        cond, body, (jnp.int32(_THRESH_BITS + 1), jnp.int32(_ONE_BITS)))
    out_ref[0, 0] = jax.lax.bitcast_convert_type(lo, jnp.float32)


def _resum_body(th_ref, p_ref, t_ref, w_ref, out_ref):
    i = pl.program_id(0)
    th = th_ref[0, 0]
    p = p_ref[...]
    loss = _bce(p, t_ref[...], w_ref[...])
    keep = p < th
    s = jnp.sum(jnp.where(keep, loss, 0.0))
    cnt = jnp.sum(keep.astype(jnp.float32))

    @pl.when(i == 0)
    def _():
        out_ref[0, 0] = 0.0
        out_ref[0, 1] = 0.0

    out_ref[0, 0] += s
    out_ref[0, 1] += cnt


def kernel(aux_out, seg_out, targets, weights):
    a4 = aux_out.reshape(2, 32, 64, 64).transpose(0, 2, 1, 3)

    sums = pl.pallas_call(
        _fused_body,
        grid=(2, _HCH),
        out_shape=jax.ShapeDtypeStruct((2, 3), jnp.float32),
        in_specs=[
            pl.BlockSpec((1, 1, 64, 64, 128), lambda b, c: (b, 0, 0, c, 0)),
            pl.BlockSpec((1, 1, 64, 64, 128), lambda b, c: (b, 0, 0, c, 0)),
            pl.BlockSpec((1, 1, 64, 64, 128), lambda b, c: (b, 0, 0, c, 0)),
            pl.BlockSpec((1, 64, 32, 64), lambda b, c: (b, 0, 0, 0)),
            pl.BlockSpec((32, 64), lambda b, c: (0, 0)),
            pl.BlockSpec((64, 128), lambda b, c: (0, 0)),
            pl.BlockSpec((128, 64), lambda b, c: (0, 0)),
        ],
        out_specs=pl.BlockSpec((2, 3), lambda b, c: (0, 0),
                               memory_space=pltpu.SMEM),
        scratch_shapes=[
            pltpu.VMEM((32, 128, 128), jnp.float32),
            pltpu.VMEM((32, 128, 128), jnp.float32),
        ],
        compiler_params=pltpu.CompilerParams(
            dimension_semantics=("arbitrary", "arbitrary")),
    )(seg_out, targets, weights, a4, jnp.asarray(_MD), jnp.asarray(_MH),
      jnp.asarray(_MW.T.copy()))

    s_fast = sums[0, 0]
    c_fast = sums[0, 1]
    aux_sum = sums[0, 2]
    c_le = sums[1, 0]

    def rare_path(_):
        p2 = seg_out.reshape(_ROWS, 1024)
        t2 = targets.reshape(_ROWS, 1024)
        w2 = weights.reshape(_ROWS, 1024)
        thresh = pl.pallas_call(
            _sel_body,
            out_shape=jax.ShapeDtypeStruct((1, 1), jnp.float32),
            in_specs=[pl.BlockSpec((_ROWS, 1024), lambda: (0, 0))],
            out_specs=pl.BlockSpec(memory_space=pltpu.SMEM),
        )(p2)
        rows_blk = 256
        seg_sums = pl.pallas_call(
            _resum_body,
            grid=(_ROWS // rows_blk,),
            out_shape=jax.ShapeDtypeStruct((1, 2), jnp.float32),
            in_specs=[
                pl.BlockSpec(memory_space=pltpu.SMEM),
                pl.BlockSpec((rows_blk, 1024), lambda i: (i, 0)),
                pl.BlockSpec((rows_blk, 1024), lambda i: (i, 0)),
                pl.BlockSpec((rows_blk, 1024), lambda i: (i, 0)),
            ],
            out_specs=pl.BlockSpec((1, 2), lambda i: (0, 0),
                                   memory_space=pltpu.SMEM),
            compiler_params=pltpu.CompilerParams(
                dimension_semantics=("arbitrary",)),
        )(thresh, p2, t2, w2)
        return seg_sums[0, 0], seg_sums[0, 1]

    def fast_path(_):
        return s_fast, c_fast

    s, c = jax.lax.cond(c_le < np.float32(_MIN_KEPT + 1),
                        rare_path, fast_path, None)
    seg_loss = s / jnp.maximum(c, 1.0)
    aux_loss = aux_sum / np.float32(_AUX_N)
    return seg_loss + 0.5 * aux_loss


# speedup vs baseline: 94.9530x; 1.0001x over previous
import numpy as np
import jax
import jax.numpy as jnp
from jax.experimental import pallas as pl
from jax.experimental.pallas import tpu as pltpu

_THRESH = np.float32(0.7)
_MIN_KEPT = 100000
_AUX_N = 2 * 1 * 32 * 64 * 64
_ROWS = 2048
_THRESH_BITS = int(np.float32(0.7).view(np.int32))
_ONE_BITS = int(np.float32(1.0).view(np.int32))
_HCH = 4


def _interp_matrix(out_size, in_size):
    pos = (np.arange(out_size, dtype=np.float32) * np.float32(in_size - 1)) \
        / np.float32(out_size - 1)
    lo = np.floor(pos).astype(np.int32)
    hi = np.minimum(lo + 1, in_size - 1)
    w = (pos - lo.astype(np.float32)).astype(np.float32)
    m = np.zeros((out_size, in_size), np.float32)
    m[np.arange(out_size), lo] += np.float32(1.0) - w
    m[np.arange(out_size), hi] += w
    return m


_MD = _interp_matrix(32, 64)
_MH = _interp_matrix(64, 128)
_MW = _interp_matrix(64, 128)


def _bce(p, t, w):
    logp = jnp.maximum(jnp.log(p), -100.0)
    log1mp = jnp.maximum(jnp.log(1.0 - p), -100.0)
    return -w * (t * logp + (1.0 - t) * log1mp)


def _fused_body(p_ref, t_ref, w_ref, a_ref, md_ref, mh_ref, mwt_ref,
                out_ref, acct_ref, accw_ref):
    b = pl.program_id(0)
    c = pl.program_id(1)
    p = p_ref[0, 0]
    t = t_ref[0, 0]
    w = w_ref[0, 0]
    loss = _bce(p, t, w)
    keep = p < _THRESH
    s = jnp.sum(jnp.where(keep, loss, 0.0))
    cnt = jnp.sum(keep.astype(jnp.float32))
    c_le = jnp.sum((p <= _THRESH).astype(jnp.float32))

    md = md_ref[...]
    pd_t = jax.lax.dot_general(md, t, (((1,), (0,)), ((), ())),
                               precision=jax.lax.Precision.DEFAULT)
    pd_w = jax.lax.dot_general(md, w, (((1,), (0,)), ((), ())),
                               precision=jax.lax.Precision.DEFAULT)
    acct_ref[:, pl.ds(c * 32, 32), :] = pd_t
    accw_ref[:, pl.ds(c * 32, 32), :] = pd_w

    @pl.when(jnp.logical_and(b == 0, c == 0))
    def _():
        out_ref[0, 0] = 0.0
        out_ref[0, 1] = 0.0
        out_ref[0, 2] = 0.0
        out_ref[1, 0] = 0.0

    out_ref[0, 0] += s
    out_ref[0, 1] += cnt
    out_ref[1, 0] += c_le

    @pl.when(c == _HCH - 1)
    def _():
        mh = mh_ref[...]
        mwt = mwt_ref[...]

        def rest(x):
            x = jax.lax.dot_general(mh, x, (((1,), (1,)), ((), ())),
                                    precision=jax.lax.Precision.DEFAULT)
            x = jax.lax.dot_general(x, mwt, (((2,), (0,)), ((), ())),
                                    precision=jax.lax.Precision.DEFAULT)
            return x

        td = rest(acct_ref[...])
        wd = rest(accw_ref[...])
        out_ref[0, 2] += jnp.sum(_bce(a_ref[0], td, wd))


def _sel_body(p_ref, out_ref):
    k1 = jnp.int32(_MIN_KEPT + 1)

    def cond(st):
        lo, hi = st
        return lo < hi

    def body(st):
        lo, hi = st
        mid = (lo + hi) // 2
        pb = jax.lax.bitcast_convert_type(p_ref[...], jnp.int32)
        cq = jnp.sum((pb <= mid).astype(jnp.int32))
        pred = cq >= k1
        return (jnp.where(pred, lo, mid + 1), jnp.where(pred, hi, mid))

    lo, _ = jax.lax.while_loop(
        cond, body, (jnp.int32(_THRESH_BITS + 1), jnp.int32(_ONE_BITS)))
    out_ref[0, 0] = jax.lax.bitcast_convert_type(lo, jnp.float32)


def _resum_body(th_ref, p_ref, t_ref, w_ref, out_ref):
    i = pl.program_id(0)
    th = th_ref[0, 0]
    p = p_ref[...]
    loss = _bce(p, t_ref[...], w_ref[...])
    keep = p < th
    s = jnp.sum(jnp.where(keep, loss, 0.0))
    cnt = jnp.sum(keep.astype(jnp.float32))

    @pl.when(i == 0)
    def _():
        out_ref[0, 0] = 0.0
        out_ref[0, 1] = 0.0

    out_ref[0, 0] += s
    out_ref[0, 1] += cnt


def kernel(aux_out, seg_out, targets, weights):
    a4 = aux_out.reshape(2, 32, 64, 64).transpose(0, 2, 1, 3)

    sums = pl.pallas_call(
        _fused_body,
        grid=(2, _HCH),
        out_shape=jax.ShapeDtypeStruct((2, 3), jnp.float32),
        in_specs=[
            pl.BlockSpec((1, 1, 64, 32, 128), lambda b, c: (b, 0, 0, c, 0)),
            pl.BlockSpec((1, 1, 64, 32, 128), lambda b, c: (b, 0, 0, c, 0)),
            pl.BlockSpec((1, 1, 64, 32, 128), lambda b, c: (b, 0, 0, c, 0)),
            pl.BlockSpec((1, 64, 32, 64), lambda b, c: (b, 0, 0, 0)),
            pl.BlockSpec((32, 64), lambda b, c: (0, 0)),
            pl.BlockSpec((64, 128), lambda b, c: (0, 0)),
            pl.BlockSpec((128, 64), lambda b, c: (0, 0)),
        ],
        out_specs=pl.BlockSpec((2, 3), lambda b, c: (0, 0),
                               memory_space=pltpu.SMEM),
        scratch_shapes=[
            pltpu.VMEM((32, 128, 128), jnp.float32),
            pltpu.VMEM((32, 128, 128), jnp.float32),
        ],
        compiler_params=pltpu.CompilerParams(
            dimension_semantics=("arbitrary", "arbitrary")),
    )(seg_out, targets, weights, a4, jnp.asarray(_MD), jnp.asarray(_MH),
      jnp.asarray(_MW.T.copy()))

    s_fast = sums[0, 0]
    c_fast = sums[0, 1]
    aux_sum = sums[0, 2]
    c_le = sums[1, 0]

    def rare_path(_):
        p2 = seg_out.reshape(_ROWS, 1024)
        t2 = targets.reshape(_ROWS, 1024)
        w2 = weights.reshape(_ROWS, 1024)
        thresh = pl.pallas_call(
            _sel_body,
            out_shape=jax.ShapeDtypeStruct((1, 1), jnp.float32),
            in_specs=[pl.BlockSpec((_ROWS, 1024), lambda: (0, 0))],
            out_specs=pl.BlockSpec(memory_space=pltpu.SMEM),
        )(p2)
        rows_blk = 256
        seg_sums = pl.pallas_call(
            _resum_body,
            grid=(_ROWS // rows_blk,),
            out_shape=jax.ShapeDtypeStruct((1, 2), jnp.float32),
            in_specs=[
                pl.BlockSpec(memory_space=pltpu.SMEM),
                pl.BlockSpec((rows_blk, 1024), lambda i: (i, 0)),
                pl.BlockSpec((rows_blk, 1024), lambda i: (i, 0)),
                pl.BlockSpec((rows_blk, 1024), lambda i: (i, 0)),
            ],
            out_specs=pl.BlockSpec((1, 2), lambda i: (0, 0),
                                   memory_space=pltpu.SMEM),
            compiler_params=pltpu.CompilerParams(
                dimension_semantics=("arbitrary",)),
        )(thresh, p2, t2, w2)
        return seg_sums[0, 0], seg_sums[0, 1]

    def fast_path(_):
        return s_fast, c_fast

    s, c = jax.lax.cond(c_le < np.float32(_MIN_KEPT + 1),
                        rare_path, fast_path, None)
    seg_loss = s / jnp.maximum(c, 1.0)
    aux_loss = aux_sum / np.float32(_AUX_N)
    return seg_loss + 0.5 * aux_loss


# speedup vs baseline: 107.0453x; 1.1274x over previous
import numpy as np
import jax
import jax.numpy as jnp
from jax.experimental import pallas as pl
from jax.experimental.pallas import tpu as pltpu

_THRESH = np.float32(0.7)
_MIN_KEPT = 100000
_AUX_N = 2 * 1 * 32 * 64 * 64
_ROWS = 2048
_THRESH_BITS = int(np.float32(0.7).view(np.int32))
_ONE_BITS = int(np.float32(1.0).view(np.int32))
_HCH = 4


def _interp_matrix(out_size, in_size):
    pos = (np.arange(out_size, dtype=np.float32) * np.float32(in_size - 1)) \
        / np.float32(out_size - 1)
    lo = np.floor(pos).astype(np.int32)
    hi = np.minimum(lo + 1, in_size - 1)
    w = (pos - lo.astype(np.float32)).astype(np.float32)
    m = np.zeros((out_size, in_size), np.float32)
    m[np.arange(out_size), lo] += np.float32(1.0) - w
    m[np.arange(out_size), hi] += w
    return m


_MD = _interp_matrix(32, 64)
_MH = _interp_matrix(64, 128)
_MW = _interp_matrix(64, 128)


def _bce(p, t, w):
    logp = jnp.maximum(jnp.log(p), -100.0)
    log1mp = jnp.maximum(jnp.log(1.0 - p), -100.0)
    return -w * (t * logp + (1.0 - t) * log1mp)


def _fused_body(p_ref, t_ref, w_ref, a_ref, md_ref, mh_ref, mwt_ref,
                out_ref, acct_ref, accw_ref):
    b = pl.program_id(0)
    c = pl.program_id(1)
    p = p_ref[0, 0]
    t = t_ref[0, 0]
    w = w_ref[0, 0]
    loss = _bce(p, t, w)
    keep = p < _THRESH
    s = jnp.sum(jnp.where(keep, loss, 0.0))
    cnt = jnp.sum(keep.astype(jnp.float32))
    c_le = jnp.sum((p <= _THRESH).astype(jnp.float32))

    md = md_ref[...]
    pd_t = jax.lax.dot_general(md, t, (((1,), (0,)), ((), ())),
                               precision=jax.lax.Precision.DEFAULT)
    pd_w = jax.lax.dot_general(md, w, (((1,), (0,)), ((), ())),
                               precision=jax.lax.Precision.DEFAULT)
    acct_ref[:, pl.ds(c * 32, 32), :] = pd_t
    accw_ref[:, pl.ds(c * 32, 32), :] = pd_w

    @pl.when(jnp.logical_and(b == 0, c == 0))
    def _():
        out_ref[0, 0] = 0.0
        out_ref[0, 1] = 0.0
        out_ref[0, 2] = 0.0
        out_ref[1, 0] = 0.0

    out_ref[0, 0] += s
    out_ref[0, 1] += cnt
    out_ref[1, 0] += c_le

    @pl.when(c == _HCH - 1)
    def _():
        mh = mh_ref[...]
        mwt = mwt_ref[...]

        def rest(x):
            x = jax.lax.dot_general(mh, x, (((1,), (1,)), ((), ())),
                                    precision=jax.lax.Precision.DEFAULT)
            x = jax.lax.dot_general(x, mwt, (((2,), (0,)), ((), ())),
                                    precision=jax.lax.Precision.DEFAULT)
            return x

        td = rest(acct_ref[...])
        wd = rest(accw_ref[...])
        a = jnp.transpose(a_ref[0, 0], (1, 0, 2))
        out_ref[0, 2] += jnp.sum(_bce(a, td, wd))


def _sel_body(p_ref, out_ref):
    k1 = jnp.int32(_MIN_KEPT + 1)

    def cond(st):
        lo, hi = st
        return lo < hi

    def body(st):
        lo, hi = st
        mid = (lo + hi) // 2
        pb = jax.lax.bitcast_convert_type(p_ref[...], jnp.int32)
        cq = jnp.sum((pb <= mid).astype(jnp.int32))
        pred = cq >= k1
        return (jnp.where(pred, lo, mid + 1), jnp.where(pred, hi, mid))

    lo, _ = jax.lax.while_loop(
        cond, body, (jnp.int32(_THRESH_BITS + 1), jnp.int32(_ONE_BITS)))
    out_ref[0, 0] = jax.lax.bitcast_convert_type(lo, jnp.float32)


def _resum_body(th_ref, p_ref, t_ref, w_ref, out_ref):
    i = pl.program_id(0)
    th = th_ref[0, 0]
    p = p_ref[...]
    loss = _bce(p, t_ref[...], w_ref[...])
    keep = p < th
    s = jnp.sum(jnp.where(keep, loss, 0.0))
    cnt = jnp.sum(keep.astype(jnp.float32))

    @pl.when(i == 0)
    def _():
        out_ref[0, 0] = 0.0
        out_ref[0, 1] = 0.0

    out_ref[0, 0] += s
    out_ref[0, 1] += cnt


def kernel(aux_out, seg_out, targets, weights):
    sums = pl.pallas_call(
        _fused_body,
        grid=(2, _HCH),
        out_shape=jax.ShapeDtypeStruct((2, 3), jnp.float32),
        in_specs=[
            pl.BlockSpec((1, 1, 64, 32, 128), lambda b, c: (b, 0, 0, c, 0)),
            pl.BlockSpec((1, 1, 64, 32, 128), lambda b, c: (b, 0, 0, c, 0)),
            pl.BlockSpec((1, 1, 64, 32, 128), lambda b, c: (b, 0, 0, c, 0)),
            pl.BlockSpec((1, 1, 32, 64, 64), lambda b, c: (b, 0, 0, 0, 0)),
            pl.BlockSpec((32, 64), lambda b, c: (0, 0)),
            pl.BlockSpec((64, 128), lambda b, c: (0, 0)),
            pl.BlockSpec((128, 64), lambda b, c: (0, 0)),
        ],
        out_specs=pl.BlockSpec((2, 3), lambda b, c: (0, 0),
                               memory_space=pltpu.SMEM),
        scratch_shapes=[
            pltpu.VMEM((32, 128, 128), jnp.float32),
            pltpu.VMEM((32, 128, 128), jnp.float32),
        ],
        compiler_params=pltpu.CompilerParams(
            dimension_semantics=("arbitrary", "arbitrary")),
    )(seg_out, targets, weights, aux_out, jnp.asarray(_MD), jnp.asarray(_MH),
      jnp.asarray(_MW.T.copy()))

    s_fast = sums[0, 0]
    c_fast = sums[0, 1]
    aux_sum = sums[0, 2]
    c_le = sums[1, 0]

    def rare_path(_):
        p2 = seg_out.reshape(_ROWS, 1024)
        t2 = targets.reshape(_ROWS, 1024)
        w2 = weights.reshape(_ROWS, 1024)
        thresh = pl.pallas_call(
            _sel_body,
            out_shape=jax.ShapeDtypeStruct((1, 1), jnp.float32),
            in_specs=[pl.BlockSpec((_ROWS, 1024), lambda: (0, 0))],
            out_specs=pl.BlockSpec(memory_space=pltpu.SMEM),
        )(p2)
        rows_blk = 256
        seg_sums = pl.pallas_call(
            _resum_body,
            grid=(_ROWS // rows_blk,),
            out_shape=jax.ShapeDtypeStruct((1, 2), jnp.float32),
            in_specs=[
                pl.BlockSpec(memory_space=pltpu.SMEM),
                pl.BlockSpec((rows_blk, 1024), lambda i: (i, 0)),
                pl.BlockSpec((rows_blk, 1024), lambda i: (i, 0)),
                pl.BlockSpec((rows_blk, 1024), lambda i: (i, 0)),
            ],
            out_specs=pl.BlockSpec((1, 2), lambda i: (0, 0),
                                   memory_space=pltpu.SMEM),
            compiler_params=pltpu.CompilerParams(
                dimension_semantics=("arbitrary",)),
        )(thresh, p2, t2, w2)
        return seg_sums[0, 0], seg_sums[0, 1]

    def fast_path(_):
        return s_fast, c_fast

    s, c = jax.lax.cond(c_le < np.float32(_MIN_KEPT + 1),
                        rare_path, fast_path, None)
    seg_loss = s / jnp.maximum(c, 1.0)
    aux_loss = aux_sum / np.float32(_AUX_N)
    return seg_loss + 0.5 * aux_loss


# speedup vs baseline: 119.6435x; 1.1177x over previous
import numpy as np
import jax
import jax.numpy as jnp
from jax.experimental import pallas as pl
from jax.experimental.pallas import tpu as pltpu

_THRESH = np.float32(0.7)
_MIN_KEPT = 100000
_AUX_N = 2 * 1 * 32 * 64 * 64
_ROWS = 2048
_THRESH_BITS = int(np.float32(0.7).view(np.int32))
_ONE_BITS = int(np.float32(1.0).view(np.int32))
_HCH = 4


def _interp_matrix(out_size, in_size):
    pos = (np.arange(out_size, dtype=np.float32) * np.float32(in_size - 1)) \
        / np.float32(out_size - 1)
    lo = np.floor(pos).astype(np.int32)
    hi = np.minimum(lo + 1, in_size - 1)
    w = (pos - lo.astype(np.float32)).astype(np.float32)
    m = np.zeros((out_size, in_size), np.float32)
    m[np.arange(out_size), lo] += np.float32(1.0) - w
    m[np.arange(out_size), hi] += w
    return m


_MD = _interp_matrix(32, 64)
_MH = _interp_matrix(64, 128)
_MW = _interp_matrix(64, 128)


def _bce(p, t, w):
    logp = jnp.maximum(jnp.log(p), -100.0)
    log1mp = jnp.maximum(jnp.log(1.0 - p), -100.0)
    return -w * (t * logp + (1.0 - t) * log1mp)


def _fused_body(p_ref, t_ref, w_ref, a_ref, md_ref, mh_ref, mwt_ref,
                out_ref, acct_ref, accw_ref):
    b = pl.program_id(0)
    c = pl.program_id(1)
    p = p_ref[0, 0]
    t = t_ref[0, 0]
    w = w_ref[0, 0]
    loss = _bce(p, t, w)
    keep = p < _THRESH
    s = jnp.sum(jnp.where(keep, loss, 0.0))
    cnt = jnp.sum(keep.astype(jnp.float32))
    c_le = jnp.sum((p <= _THRESH).astype(jnp.float32))

    md = md_ref[...]
    pd_t = jax.lax.dot_general(md, t, (((1,), (0,)), ((), ())),
                               precision=jax.lax.Precision.DEFAULT)
    pd_w = jax.lax.dot_general(md, w, (((1,), (0,)), ((), ())),
                               precision=jax.lax.Precision.DEFAULT)
    acct_ref[:, pl.ds(c * 32, 32), :] = pd_t
    accw_ref[:, pl.ds(c * 32, 32), :] = pd_w

    @pl.when(jnp.logical_and(b == 0, c == 0))
    def _():
        out_ref[0, 0] = 0.0
        out_ref[0, 1] = 0.0
        out_ref[0, 2] = 0.0
        out_ref[1, 0] = 0.0

    out_ref[0, 0] += s
    out_ref[0, 1] += cnt
    out_ref[1, 0] += c_le

    @pl.when(c == _HCH - 1)
    def _():
        mh = mh_ref[...]
        mwt = mwt_ref[...]

        def rest(x):
            x = jax.lax.dot_general(mh, x, (((1,), (1,)), ((), ())),
                                    precision=jax.lax.Precision.DEFAULT)
            x = jax.lax.dot_general(x, mwt, (((2,), (0,)), ((), ())),
                                    precision=jax.lax.Precision.DEFAULT)
            return x

        td = rest(acct_ref[...])
        wd = rest(accw_ref[...])
        a = jnp.transpose(a_ref[0, 0], (1, 0, 2))
        out_ref[0, 2] += jnp.sum(_bce(a, td, wd))

    @pl.when(jnp.logical_and(b == 1, c == _HCH - 1))
    def _():
        seg_fast = out_ref[0, 0] / jnp.maximum(out_ref[0, 1], 1.0)
        out_ref[1, 1] = seg_fast + 0.5 * (out_ref[0, 2] / np.float32(_AUX_N))


def _sel_body(p_ref, out_ref):
    k1 = jnp.int32(_MIN_KEPT + 1)

    def cond(st):
        lo, hi = st
        return lo < hi

    def body(st):
        lo, hi = st
        mid = (lo + hi) // 2
        pb = jax.lax.bitcast_convert_type(p_ref[...], jnp.int32)
        cq = jnp.sum((pb <= mid).astype(jnp.int32))
        pred = cq >= k1
        return (jnp.where(pred, lo, mid + 1), jnp.where(pred, hi, mid))

    lo, _ = jax.lax.while_loop(
        cond, body, (jnp.int32(_THRESH_BITS + 1), jnp.int32(_ONE_BITS)))
    out_ref[0, 0] = jax.lax.bitcast_convert_type(lo, jnp.float32)


def _resum_body(th_ref, p_ref, t_ref, w_ref, out_ref):
    i = pl.program_id(0)
    th = th_ref[0, 0]
    p = p_ref[...]
    loss = _bce(p, t_ref[...], w_ref[...])
    keep = p < th
    s = jnp.sum(jnp.where(keep, loss, 0.0))
    cnt = jnp.sum(keep.astype(jnp.float32))

    @pl.when(i == 0)
    def _():
        out_ref[0, 0] = 0.0
        out_ref[0, 1] = 0.0

    out_ref[0, 0] += s
    out_ref[0, 1] += cnt


def kernel(aux_out, seg_out, targets, weights):
    sums = pl.pallas_call(
        _fused_body,
        grid=(2, _HCH),
        out_shape=jax.ShapeDtypeStruct((2, 3), jnp.float32),
        in_specs=[
            pl.BlockSpec((1, 1, 64, 32, 128), lambda b, c: (b, 0, 0, c, 0)),
            pl.BlockSpec((1, 1, 64, 32, 128), lambda b, c: (b, 0, 0, c, 0)),
            pl.BlockSpec((1, 1, 64, 32, 128), lambda b, c: (b, 0, 0, c, 0)),
            pl.BlockSpec((1, 1, 32, 64, 64), lambda b, c: (b, 0, 0, 0, 0)),
            pl.BlockSpec((32, 64), lambda b, c: (0, 0)),
            pl.BlockSpec((64, 128), lambda b, c: (0, 0)),
            pl.BlockSpec((128, 64), lambda b, c: (0, 0)),
        ],
        out_specs=pl.BlockSpec((2, 3), lambda b, c: (0, 0),
                               memory_space=pltpu.SMEM),
        scratch_shapes=[
            pltpu.VMEM((32, 128, 128), jnp.float32),
            pltpu.VMEM((32, 128, 128), jnp.float32),
        ],
        compiler_params=pltpu.CompilerParams(
            dimension_semantics=("arbitrary", "arbitrary")),
    )(seg_out, targets, weights, aux_out, jnp.asarray(_MD), jnp.asarray(_MH),
      jnp.asarray(_MW.T.copy()))

    fast_total = sums[1, 1]
    aux_sum = sums[0, 2]
    c_le = sums[1, 0]

    def rare_path(_):
        p2 = seg_out.reshape(_ROWS, 1024)
        t2 = targets.reshape(_ROWS, 1024)
        w2 = weights.reshape(_ROWS, 1024)
        thresh = pl.pallas_call(
            _sel_body,
            out_shape=jax.ShapeDtypeStruct((1, 1), jnp.float32),
            in_specs=[pl.BlockSpec((_ROWS, 1024), lambda: (0, 0))],
            out_specs=pl.BlockSpec(memory_space=pltpu.SMEM),
        )(p2)
        rows_blk = 256
        seg_sums = pl.pallas_call(
            _resum_body,
            grid=(_ROWS // rows_blk,),
            out_shape=jax.ShapeDtypeStruct((1, 2), jnp.float32),
            in_specs=[
                pl.BlockSpec(memory_space=pltpu.SMEM),
                pl.BlockSpec((rows_blk, 1024), lambda i: (i, 0)),
                pl.BlockSpec((rows_blk, 1024), lambda i: (i, 0)),
                pl.BlockSpec((rows_blk, 1024), lambda i: (i, 0)),
            ],
            out_specs=pl.BlockSpec((1, 2), lambda i: (0, 0),
                                   memory_space=pltpu.SMEM),
            compiler_params=pltpu.CompilerParams(
                dimension_semantics=("arbitrary",)),
        )(thresh, p2, t2, w2)
        seg_loss = seg_sums[0, 0] / jnp.maximum(seg_sums[0, 1], 1.0)
        return seg_loss + 0.5 * (aux_sum / np.float32(_AUX_N))

    def fast_path(_):
        return fast_total

    return jax.lax.cond(c_le < np.float32(_MIN_KEPT + 1),
                        rare_path, fast_path, None)
